# Initial kernel scaffold; baseline (speedup 1.0000x reference)
import jax
import jax.numpy as jnp
from jax.experimental import pallas as pl


def kernel(u, v, adj_ent, adj_rel, usr_emb, ent_emb, rel_emb, W, b):
    raise NotImplementedError("write your pallas kernel here")


# speedup vs baseline: 1.0464x; 1.0464x over previous
import functools

import jax
import jax.numpy as jnp
from jax import lax
from jax.experimental import pallas as pl
from jax.experimental.pallas import tpu as pltpu
from jax.experimental.pallas import tpu_sc as plsc

B = 1024
K = 16
D = 32
NUM_REL = 32

NC = 2
NS = 16
NW = NC * NS
BPW = B // NW
IDX_CHUNK = 128
NCHUNK = (BPW * K) // IDX_CHUNK


def _sc_gather_kernel():
  mesh = plsc.VectorSubcoreMesh(
      core_axis_name="c", subcore_axis_name="s",
      num_cores=NC, num_subcores=NS)

  @functools.partial(
      pl.kernel,
      mesh=mesh,
      compiler_params=pltpu.CompilerParams(use_tc_tiling_on_sc=False),
      out_type=(
          jax.ShapeDtypeStruct((B, D), jnp.float32),
          jax.ShapeDtypeStruct((B, D), jnp.float32),
          jax.ShapeDtypeStruct((B * K, D), jnp.float32),
          jax.ShapeDtypeStruct((B, K), jnp.int32),
      ),
      scratch_types=[
          pltpu.VMEM((BPW,), jnp.int32),
          pltpu.VMEM((BPW,), jnp.int32),
          pltpu.VMEM((BPW, D), jnp.float32),
          pltpu.VMEM((BPW, D), jnp.float32),
          pltpu.VMEM((BPW, K), jnp.int32),
          pltpu.VMEM((BPW, K), jnp.int32),
          pltpu.VMEM((NCHUNK, IDX_CHUNK), jnp.int32),
          pltpu.VMEM((BPW * K, D), jnp.float32),
          pltpu.SemaphoreType.DMA,
          pltpu.SemaphoreType.DMA,
          pltpu.SemaphoreType.DMA,
          pltpu.SemaphoreType.DMA,
          pltpu.SemaphoreType.DMA,
      ],
  )
  def sc_gather(u_h, v_h, adj_ent_h, adj_rel_h, usr_h, ent_h,
                ue_o, vs_o, ne_o, rel_o,
                uix, vix, uev, vsv, nbrv, relv, nflat, nev,
                sem_ue, sem_nb, sem_re, sem_vs, sem_ne):
    wid = lax.axis_index("s") * NC + lax.axis_index("c")
    base = wid * BPW
    pltpu.sync_copy(u_h.at[pl.ds(base, BPW)], uix)
    pltpu.sync_copy(v_h.at[pl.ds(base, BPW)], vix)
    c_ue = pltpu.async_copy(usr_h.at[uix], uev, sem_ue)
    c_nb = pltpu.async_copy(adj_ent_h.at[vix], nbrv, sem_nb)
    c_re = pltpu.async_copy(adj_rel_h.at[vix], relv, sem_re)
    c_vs = pltpu.async_copy(ent_h.at[vix], vsv, sem_vs)
    c_nb.wait()
    for j in range(BPW):
      row = nbrv[j]
      nflat[(j * K) // IDX_CHUNK, pl.ds((j * K) % IDX_CHUNK, K)] = row
    chained = [
        pltpu.async_copy(ent_h.at[nflat.at[c]],
                         nev.at[pl.ds(c * IDX_CHUNK, IDX_CHUNK)], sem_ne)
        for c in range(NCHUNK)
    ]
    c_ue.wait()
    pltpu.sync_copy(uev, ue_o.at[pl.ds(base, BPW)])
    c_re.wait()
    pltpu.sync_copy(relv, rel_o.at[pl.ds(base, BPW)])
    c_vs.wait()
    pltpu.sync_copy(vsv, vs_o.at[pl.ds(base, BPW)])
    for cp in chained:
      cp.wait()
    pltpu.sync_copy(nev, ne_o.at[pl.ds(base * K, BPW * K)])

  return sc_gather


_RB = 128
_GB = B // _RB


def _tc_dense(ue_r, vs_r, ne_r, rel_r, relemb_t_r, wt_r, b_r, out_r):
  ue = ue_r[...]
  logits_all = jnp.dot(ue, relemb_t_r[...],
                       preferred_element_type=jnp.float32,
                       precision=lax.Precision.HIGHEST)
  rel = rel_r[...]
  riota = lax.broadcasted_iota(jnp.int32, (_RB, K, NUM_REL), 2)
  onehot = riota == rel[:, :, None]
  logits = jnp.sum(jnp.where(onehot, logits_all[:, None, :], 0.0), axis=2)
  m = jnp.max(logits, axis=1, keepdims=True)
  e = jnp.exp(logits - m)
  p = e / jnp.sum(e, axis=1, keepdims=True)
  ne = ne_r[...]
  e_u = jnp.sum(ne * p[:, :, None], axis=1)
  x = e_u + vs_r[...]
  vu = jnp.dot(x, wt_r[...], preferred_element_type=jnp.float32,
               precision=lax.Precision.HIGHEST) + b_r[...]
  vu = jnp.maximum(vu, 0.0)
  y = jnp.sum(ue * vu, axis=1)
  out_r[...] = (1.0 / (1.0 + jnp.exp(-y)))[:, None]


def kernel(u, v, adj_ent, adj_rel, usr_emb, ent_emb, rel_emb, W, b):
  ue, vs, ne, rel = _sc_gather_kernel()(
      u.astype(jnp.int32), v.astype(jnp.int32),
      adj_ent.astype(jnp.int32), adj_rel.astype(jnp.int32),
      usr_emb, ent_emb)
  ne3 = ne.reshape(B, K, D)
  out = pl.pallas_call(
      _tc_dense,
      grid=(_GB,),
      in_specs=[
          pl.BlockSpec((_RB, D), lambda i: (i, 0)),
          pl.BlockSpec((_RB, D), lambda i: (i, 0)),
          pl.BlockSpec((_RB, K, D), lambda i: (i, 0, 0)),
          pl.BlockSpec((_RB, K), lambda i: (i, 0)),
          pl.BlockSpec((D, NUM_REL), lambda i: (0, 0)),
          pl.BlockSpec((D, D), lambda i: (0, 0)),
          pl.BlockSpec((1, D), lambda i: (0, 0)),
      ],
      out_specs=pl.BlockSpec((_RB, 1), lambda i: (i, 0)),
      out_shape=jax.ShapeDtypeStruct((B, 1), jnp.float32),
  )(ue, vs, ne3, rel, rel_emb.T, W.T, b.reshape(1, D))
  return out.reshape(B)


# speedup vs baseline: 1.9173x; 1.8323x over previous
import functools

import jax
import jax.numpy as jnp
from jax import lax
from jax.experimental import pallas as pl
from jax.experimental.pallas import tpu as pltpu
from jax.experimental.pallas import tpu_sc as plsc

B = 1024
K = 16
D = 32
NUM_REL = 32
NUM_ENT = 100000
NUM_USR = 10000

NC = 2
NS = 16
NW = NC * NS
BPW = B // NW
TPW = BPW // 8


def _sc_gather_kernel():
  mesh = plsc.VectorSubcoreMesh(
      core_axis_name="c", subcore_axis_name="s",
      num_cores=NC, num_subcores=NS)

  @functools.partial(
      pl.kernel,
      mesh=mesh,
      compiler_params=pltpu.CompilerParams(use_tc_tiling_on_sc=True),
      out_type=(
          jax.ShapeDtypeStruct((B // 8, 8, D), jnp.float32),
          jax.ShapeDtypeStruct((B // 8, 8, D), jnp.float32),
          jax.ShapeDtypeStruct((B * K // 8, 8, D), jnp.float32),
          jax.ShapeDtypeStruct((B // 8, 8, K), jnp.int32),
      ),
      scratch_types=[
          pltpu.VMEM((BPW,), jnp.int32),
          pltpu.VMEM((BPW,), jnp.int32),
          pltpu.VMEM((TPW, 8, K), jnp.int32),
          pltpu.VMEM((TPW, 8, D), jnp.float32),
          pltpu.VMEM((TPW, 8, D), jnp.float32),
          pltpu.VMEM((TPW, 8, K), jnp.int32),
          pltpu.VMEM((BPW * K // 8, 8, D), jnp.float32),
          pltpu.SemaphoreType.DMA,
          pltpu.SemaphoreType.DMA,
      ],
  )
  def sc_gather(u_h, v_h, ae3, ar3, usr3, ent3,
                ue_o, vs_o, ne_o, rel_o,
                uixv, vixv, nbrv3, uev, vsv, relv, nev,
                sem_r, sem_a):
    wid = lax.axis_index("s") * NC + lax.axis_index("c")
    base = wid * BPW
    pltpu.sync_copy(u_h.at[pl.ds(base, BPW)], uixv)
    pltpu.sync_copy(v_h.at[pl.ds(base, BPW)], vixv)
    for c in range(BPW // 16):
      uvec = uixv[pl.ds(c * 16, 16)]
      vvec = vixv[pl.ds(c * 16, 16)]
      for l in range(16):
        j = c * 16 + l
        tj, sj = j >> 3, j & 7
        vv = vvec[l]
        vt, vs2 = vv >> 3, vv & 7
        uu = uvec[l]
        pltpu.async_copy(ae3.at[vt, vs2], nbrv3.at[tj, sj], sem_a)
        pltpu.async_copy(ar3.at[vt, vs2], relv.at[tj, sj], sem_a)
        pltpu.async_copy(usr3.at[uu >> 3, uu & 7], uev.at[tj, sj], sem_r)
        pltpu.async_copy(ent3.at[vt, vs2], vsv.at[tj, sj], sem_r)
    for j in range(BPW):
      tj, sj = j >> 3, j & 7
      pltpu.make_async_copy(ae3.at[0, 0], nbrv3.at[tj, sj], sem_a).wait()
      pltpu.make_async_copy(ar3.at[0, 0], relv.at[tj, sj], sem_a).wait()
    def issue_ne(j, carry):
      row = nbrv3[j >> 3, j & 7]
      for k in range(K):
        e = row[k]
        pltpu.async_copy(ent3.at[e >> 3, e & 7],
                         nev.at[2 * j + k // 8, k % 8], sem_r)
      return carry
    lax.fori_loop(0, BPW, issue_ne, 0)
    def drain_ne(j, carry):
      for k in range(K):
        pltpu.make_async_copy(ent3.at[0, 0],
                              nev.at[2 * j + k // 8, k % 8],
                              sem_r).wait()
      return carry
    for j in range(BPW):
      tj, sj = j >> 3, j & 7
      pltpu.make_async_copy(usr3.at[0, 0], uev.at[tj, sj], sem_r).wait()
      pltpu.make_async_copy(ent3.at[0, 0], vsv.at[tj, sj], sem_r).wait()
    lax.fori_loop(0, BPW, drain_ne, 0)
    pltpu.sync_copy(uev, ue_o.at[pl.ds(wid * TPW, TPW)])
    pltpu.sync_copy(vsv, vs_o.at[pl.ds(wid * TPW, TPW)])
    pltpu.sync_copy(relv, rel_o.at[pl.ds(wid * TPW, TPW)])
    pltpu.sync_copy(nev, ne_o.at[pl.ds(wid * TPW * K, TPW * K)])

  return sc_gather


_RB = 128
_GB = B // _RB


def _tc_dense(ue_r, vs_r, ne_r, rel_r, relemb_t_r, wt_r, b_r, out_r):
  ue = ue_r[...]
  logits_all = jnp.dot(ue, relemb_t_r[...],
                       preferred_element_type=jnp.float32,
                       precision=lax.Precision.HIGHEST)
  rel = rel_r[...]
  riota = lax.broadcasted_iota(jnp.int32, (_RB, K, NUM_REL), 2)
  onehot = riota == rel[:, :, None]
  logits = jnp.sum(jnp.where(onehot, logits_all[:, None, :], 0.0), axis=2)
  m = jnp.max(logits, axis=1, keepdims=True)
  e = jnp.exp(logits - m)
  p = e / jnp.sum(e, axis=1, keepdims=True)
  ne = ne_r[...]
  e_u = jnp.sum(ne * p[:, :, None], axis=1)
  x = e_u + vs_r[...]
  vu = jnp.dot(x, wt_r[...], preferred_element_type=jnp.float32,
               precision=lax.Precision.HIGHEST) + b_r[...]
  vu = jnp.maximum(vu, 0.0)
  y = jnp.sum(ue * vu, axis=1)
  out_r[...] = (1.0 / (1.0 + jnp.exp(-y)))[:, None]


def kernel(u, v, adj_ent, adj_rel, usr_emb, ent_emb, rel_emb, W, b):
  ae3 = adj_ent.reshape(NUM_ENT // 8, 8, K)
  ar3 = adj_rel.reshape(NUM_ENT // 8, 8, K)
  usr3 = usr_emb.reshape(NUM_USR // 8, 8, D)
  ent3 = ent_emb.reshape(NUM_ENT // 8, 8, D)
  ue, vs, ne, rel = _sc_gather_kernel()(
      u.astype(jnp.int32), v.astype(jnp.int32), ae3, ar3, usr3, ent3)
  out = pl.pallas_call(
      _tc_dense,
      grid=(_GB,),
      in_specs=[
          pl.BlockSpec((_RB, D), lambda i: (i, 0)),
          pl.BlockSpec((_RB, D), lambda i: (i, 0)),
          pl.BlockSpec((_RB, K, D), lambda i: (i, 0, 0)),
          pl.BlockSpec((_RB, K), lambda i: (i, 0)),
          pl.BlockSpec((D, NUM_REL), lambda i: (0, 0)),
          pl.BlockSpec((D, D), lambda i: (0, 0)),
          pl.BlockSpec((1, D), lambda i: (0, 0)),
      ],
      out_specs=pl.BlockSpec((_RB, 1), lambda i: (i, 0)),
      out_shape=jax.ShapeDtypeStruct((B, 1), jnp.float32),
  )(ue.reshape(B, D), vs.reshape(B, D), ne.reshape(B, K, D),
    rel.reshape(B, K), rel_emb.T, W.T, b.reshape(1, D))
  return out.reshape(B)
